# baseline (speedup 1.0000x reference)
import functools

import jax
import jax.numpy as jnp
from jax import lax
from jax.experimental import pallas as pl
from jax.experimental.pallas import tpu as pltpu
from jax.experimental.pallas import tpu_sc as plsc

EMBED = 64
SCALE = 8.0

_info = plsc.get_sparse_core_info()
_NC, _NS, _L = _info.num_cores, _info.num_subcores, _info.num_lanes
_NW = _NC * _NS

_TBLK = 512


def _rowmajor_scaled_table(table_t):
    emb, vocab = table_t.shape
    grid = (vocab + _TBLK - 1) // _TBLK

    def body(x_ref, o_ref):
        xt = (x_ref[...] * SCALE).T
        o_ref[...] = jnp.concatenate(
            [xt, jnp.zeros((_TBLK, emb), jnp.float32)], axis=1
        )

    return pl.pallas_call(
        body,
        grid=(grid,),
        in_specs=[pl.BlockSpec((emb, _TBLK), lambda c: (0, c))],
        out_specs=pl.BlockSpec((_TBLK, 2 * emb), lambda c: (c, 0)),
        out_shape=jax.ShapeDtypeStruct((vocab, 2 * emb), jnp.float32),
    )(table_t)


def _gather_to_layout(xt_flat, table_rm, n_rows, row_len):
    blk = 128
    n_it = n_rows // blk

    @functools.partial(
        pl.kernel,
        out_type=jax.ShapeDtypeStruct(
            (row_len, EMBED // 8, n_it, 8, blk, 1), jnp.float32
        ),
        mesh=plsc.VectorSubcoreMesh(core_axis_name="c", subcore_axis_name="s"),
        scratch_types=[
            pltpu.VMEM((blk,), jnp.int32),
            pltpu.VMEM((blk, 2 * EMBED), jnp.float32),
            pltpu.SemaphoreType.DMA,
            pltpu.SemaphoreType.DMA,
        ],
        compiler_params=pltpu.CompilerParams(
            use_tc_tiling_on_sc=False, needs_layout_passes=False
        ),
    )
    def k2(xt_hbm, tbl_hbm, z_hbm, idx_v, rows_v, gsem, wsem):
        wid = lax.axis_index("s") * _NC + lax.axis_index("c")

        def j_body(j, carry):
            pltpu.sync_copy(
                xt_hbm.at[pl.ds(j * n_rows + wid * blk, blk)], idx_v
            )
            pltpu.async_copy(tbl_hbm.at[idx_v], rows_v, gsem).wait()
            handles = []
            for d in range(EMBED):
                handles.append(
                    pltpu.async_copy(
                        rows_v.at[:, pl.ds(d, 1)],
                        z_hbm.at[j, d // 8, wid, d % 8],
                        wsem,
                    )
                )
            for h in handles:
                h.wait()
            return carry

        lax.fori_loop(0, row_len, j_body, 0)

    return k2(xt_flat, table_rm)


@functools.partial(jax.jit, static_argnums=(2, 3))
def _lookup(xt_flat, table_t, n_rows, row_len):
    table128 = _rowmajor_scaled_table(table_t)
    table128 = table128.reshape(-1).reshape(table128.shape)
    z = _gather_to_layout(xt_flat, table128, n_rows, row_len)
    z = z.reshape(row_len, EMBED // 8, n_rows // 128, 8, 128)
    out = z.transpose(2, 4, 0, 1, 3).reshape(n_rows, row_len, EMBED)
    return out


def kernel(x, embedding_table):
    n_rows, row_len = x.shape
    xt_flat = x.T.reshape(-1).astype(jnp.int32)
    return _lookup(xt_flat, embedding_table.T, n_rows, row_len)


# speedup vs baseline: 94.0082x; 94.0082x over previous
import functools

import jax
import jax.numpy as jnp
from jax import lax
from jax.experimental import pallas as pl
from jax.experimental.pallas import tpu as pltpu
from jax.experimental.pallas import tpu_sc as plsc

EMBED = 64
SCALE = 8.0

_info = plsc.get_sparse_core_info()
_NC, _NS, _L = _info.num_cores, _info.num_subcores, _info.num_lanes
_NW = _NC * _NS


@functools.partial(jax.jit, static_argnames=("b_per_w", "chunk"))
def _lookup(x_flat, table, b_per_w, chunk):
    n_chunks = b_per_w // chunk
    mesh = plsc.VectorSubcoreMesh(core_axis_name="c", subcore_axis_name="s")

    @functools.partial(
        pl.kernel,
        out_type=jax.ShapeDtypeStruct((x_flat.shape[0], EMBED), jnp.float32),
        mesh=mesh,
        scratch_types=[
            pltpu.VMEM((chunk,), jnp.int32),
            pltpu.VMEM((chunk,), jnp.int32),
            pltpu.VMEM((chunk, EMBED), jnp.float32),
            pltpu.VMEM((chunk, EMBED), jnp.float32),
            pltpu.SemaphoreType.DMA,
            pltpu.SemaphoreType.DMA,
        ],
        compiler_params=pltpu.CompilerParams(use_tc_tiling_on_sc=False),
    )
    def k(x_hbm, table_hbm, out_hbm, idx0, idx1, rows0, rows1, sem0, sem1):
        wid = lax.axis_index("s") * _NC + lax.axis_index("c")
        base = wid * b_per_w
        idx_v = (idx0, idx1)
        rows_v = (rows0, rows1)
        sems = (sem0, sem1)

        def start_gather(g, b):
            off = base + g * chunk
            pltpu.sync_copy(x_hbm.at[pl.ds(off, chunk)], idx_v[b])
            return pltpu.async_copy(table_hbm.at[idx_v[b]], rows_v[b], sems[b])

        def scale_and_store(g, b):
            rv = rows_v[b]

            def scale_row(r, c2):
                for c in range(EMBED // _L):
                    sl = pl.ds(c * _L, _L)
                    rv[r, sl] = rv[r, sl] * SCALE
                return c2

            lax.fori_loop(0, chunk, scale_row, 0)
            pltpu.sync_copy(rv, out_hbm.at[pl.ds(base + g * chunk, chunk)])

        start_gather(0, 0)

        def pair_body(t, carry):
            for b in range(2):
                g = 2 * t + b
                pltpu.make_async_copy(
                    table_hbm.at[idx_v[b]], rows_v[b], sems[b]
                ).wait()

                @pl.when(g + 1 < n_chunks)
                def _prefetch():
                    start_gather(g + 1, 1 - b)

                scale_and_store(g, b)
            return carry

        lax.fori_loop(0, n_chunks // 2, pair_body, 0)

    return k(x_flat, table)


def kernel(x, embedding_table):
    orig_shape = x.shape
    x_flat = x.reshape(-1).astype(jnp.int32)
    b = x_flat.shape[0]
    b_per_w = b // _NW
    chunk = 512
    assert b_per_w % (2 * chunk) == 0
    out = _lookup(x_flat, embedding_table, b_per_w, chunk)
    return out.reshape(*orig_shape, EMBED)


# speedup vs baseline: 98.6227x; 1.0491x over previous
import functools

import jax
import jax.numpy as jnp
from jax import lax
from jax.experimental import pallas as pl
from jax.experimental.pallas import tpu as pltpu
from jax.experimental.pallas import tpu_sc as plsc

EMBED = 64
SCALE = 8.0

_info = plsc.get_sparse_core_info()
_NC, _NS, _L = _info.num_cores, _info.num_subcores, _info.num_lanes
_NW = _NC * _NS


@functools.partial(jax.jit, static_argnames=("b_per_w", "chunk"))
def _lookup(x_flat, table, b_per_w, chunk):
    n_chunks = b_per_w // chunk
    mesh = plsc.VectorSubcoreMesh(core_axis_name="c", subcore_axis_name="s")

    @functools.partial(
        pl.kernel,
        out_type=jax.ShapeDtypeStruct((x_flat.shape[0], EMBED), jnp.float32),
        mesh=mesh,
        scratch_types=[
            pltpu.VMEM((chunk,), jnp.int32),
            pltpu.VMEM((chunk,), jnp.int32),
            pltpu.VMEM((chunk, EMBED), jnp.float32),
            pltpu.VMEM((chunk, EMBED), jnp.float32),
            pltpu.SemaphoreType.DMA,
            pltpu.SemaphoreType.DMA,
        ],
        compiler_params=pltpu.CompilerParams(use_tc_tiling_on_sc=False),
    )
    def k(x_hbm, table_hbm, out_hbm, idx0, idx1, rows0, rows1, sem0, sem1):
        wid = lax.axis_index("s") * _NC + lax.axis_index("c")
        base = wid * b_per_w
        idx_v = (idx0, idx1)
        rows_v = (rows0, rows1)
        sems = (sem0, sem1)

        def start_gather(g, b):
            off = base + g * chunk
            pltpu.sync_copy(x_hbm.at[pl.ds(off, chunk)], idx_v[b])
            return pltpu.async_copy(table_hbm.at[idx_v[b]], rows_v[b], sems[b])

        def scale_and_store(g, b):
            rv = rows_v[b]

            def scale_rows(r2, c2):
                for u in range(2):
                    for c in range(EMBED // _L):
                        sl = pl.ds(c * _L, _L)
                        rv[2 * r2 + u, sl] = rv[2 * r2 + u, sl] * SCALE
                return c2

            lax.fori_loop(0, chunk // 2, scale_rows, 0)
            pltpu.sync_copy(rv, out_hbm.at[pl.ds(base + g * chunk, chunk)])

        start_gather(0, 0)

        def pair_body(t, carry):
            for b in range(2):
                g = 2 * t + b
                pltpu.make_async_copy(
                    table_hbm.at[idx_v[b]], rows_v[b], sems[b]
                ).wait()

                @pl.when(g + 1 < n_chunks)
                def _prefetch():
                    start_gather(g + 1, 1 - b)

                scale_and_store(g, b)
            return carry

        lax.fori_loop(0, n_chunks // 2, pair_body, 0)

    return k(x_flat, table)


def kernel(x, embedding_table):
    orig_shape = x.shape
    x_flat = x.reshape(-1).astype(jnp.int32)
    b = x_flat.shape[0]
    b_per_w = b // _NW
    chunk = 640
    assert b_per_w % (2 * chunk) == 0
    out = _lookup(x_flat, embedding_table, b_per_w, chunk)
    return out.reshape(*orig_shape, EMBED)


# speedup vs baseline: 98.7253x; 1.0010x over previous
import functools

import jax
import jax.numpy as jnp
from jax import lax
from jax.experimental import pallas as pl
from jax.experimental.pallas import tpu as pltpu
from jax.experimental.pallas import tpu_sc as plsc

EMBED = 64
SCALE = 8.0

_info = plsc.get_sparse_core_info()
_NC, _NS, _L = _info.num_cores, _info.num_subcores, _info.num_lanes
_NW = _NC * _NS


@functools.partial(jax.jit, static_argnames=("b_per_w", "chunk"))
def _lookup(x_flat, table, b_per_w, chunk):
    n_chunks = b_per_w // chunk
    mesh = plsc.VectorSubcoreMesh(core_axis_name="c", subcore_axis_name="s")

    @functools.partial(
        pl.kernel,
        out_type=jax.ShapeDtypeStruct((x_flat.shape[0], EMBED), jnp.float32),
        mesh=mesh,
        scratch_types=[
            pltpu.VMEM((chunk,), jnp.int32),
            pltpu.VMEM((chunk,), jnp.int32),
            pltpu.VMEM((chunk, EMBED), jnp.float32),
            pltpu.VMEM((chunk, EMBED), jnp.float32),
            pltpu.SemaphoreType.DMA,
            pltpu.SemaphoreType.DMA,
        ],
        compiler_params=pltpu.CompilerParams(use_tc_tiling_on_sc=False),
    )
    def k(x_hbm, table_hbm, out_hbm, idx0, idx1, rows0, rows1, sem0, sem1):
        wid = lax.axis_index("s") * _NC + lax.axis_index("c")
        base = wid * b_per_w
        idx_v = (idx0, idx1)
        rows_v = (rows0, rows1)
        sems = (sem0, sem1)

        def start_gather(g, b):
            off = base + g * chunk
            pltpu.sync_copy(x_hbm.at[pl.ds(off, chunk)], idx_v[b])
            return pltpu.async_copy(table_hbm.at[idx_v[b]], rows_v[b], sems[b])

        def scale_and_store(g, b):
            rv = rows_v[b]

            def scale_rows(r2, c2):
                for u in range(4):
                    for c in range(EMBED // _L):
                        sl = pl.ds(c * _L, _L)
                        rv[4 * r2 + u, sl] = rv[4 * r2 + u, sl] * SCALE
                return c2

            lax.fori_loop(0, chunk // 4, scale_rows, 0)
            pltpu.sync_copy(rv, out_hbm.at[pl.ds(base + g * chunk, chunk)])

        start_gather(0, 0)

        def pair_body(t, carry):
            for b in range(2):
                g = 2 * t + b
                pltpu.make_async_copy(
                    table_hbm.at[idx_v[b]], rows_v[b], sems[b]
                ).wait()

                @pl.when(g + 1 < n_chunks)
                def _prefetch():
                    start_gather(g + 1, 1 - b)

                scale_and_store(g, b)
            return carry

        lax.fori_loop(0, n_chunks // 2, pair_body, 0)

    return k(x_flat, table)


def kernel(x, embedding_table):
    orig_shape = x.shape
    x_flat = x.reshape(-1).astype(jnp.int32)
    b = x_flat.shape[0]
    b_per_w = b // _NW
    chunk = 800
    assert b_per_w % (2 * chunk) == 0
    out = _lookup(x_flat, embedding_table, b_per_w, chunk)
    return out.reshape(*orig_shape, EMBED)
